# baseline (speedup 1.0000x reference)
import jax
import jax.numpy as jnp
from jax.experimental import pallas as pl
from jax.experimental.pallas import tpu as pltpu

SIDE = 28
N = SIDE * SIDE
NPAD = 1024
CARD = 50
CHUNK = 128


def _tc_body(p_ref, I_ref, J_ref, dgm1_ref, dgm2_ref):
    p = p_ref[...]

    m = jax.lax.broadcasted_iota(jnp.int32, (1, 128), 1)
    k = m // 2
    tgt = jnp.where(m % 2 == 0, k, (N - 1) - k).astype(jnp.float32)

    ii = jax.lax.broadcasted_iota(jnp.int32, (N, CHUNK), 0)
    jj0 = jax.lax.broadcasted_iota(jnp.int32, (N, CHUNK), 1)
    d = ii - jj0
    jlt = [d > (c * CHUNK) for c in range(NPAD // CHUNK)]

    for mat_ref, out_ref in ((I_ref, dgm1_ref), (J_ref, dgm2_ref)):
        x = jax.lax.dot_general(
            mat_ref[...], p,
            dimension_numbers=(((1,), (0,)), ((), ())),
            preferred_element_type=jnp.float32,
        )
        xrow = x.reshape(1, N)
        xrow = jnp.concatenate(
            [xrow, jnp.full((1, NPAD - N), jnp.inf, jnp.float32)], axis=1)
        acc = None
        for c in range(NPAD // CHUNK):
            xj = jax.lax.slice(xrow, (0, c * CHUNK), (1, (c + 1) * CHUNK))
            lt = (xj < x)
            eq_lo = (xj == x) & jlt[c]
            cnt = (lt | eq_lo).astype(jnp.float32)
            acc = cnt if acc is None else acc + cnt
        rank = jax.lax.dot_general(
            acc, jnp.ones((CHUNK, 1), jnp.float32),
            dimension_numbers=(((1,), (0,)), ((), ())),
            preferred_element_type=jnp.float32,
        )
        onehot = (rank == tgt).astype(jnp.float32)
        vals = jax.lax.dot_general(
            onehot, x,
            dimension_numbers=(((0,), (0,)), ((), ())),
            preferred_element_type=jnp.float32,
        )
        out_ref[...] = vals[: 2 * CARD].reshape(CARD, 2)


def kernel(p, I, J):
    p2 = p.reshape(N, 1)
    out_sd = jax.ShapeDtypeStruct((CARD, 2), jnp.float32)
    dgm1, dgm2 = pl.pallas_call(
        _tc_body,
        out_shape=(out_sd, out_sd),
    )(p2, I, J)
    return (dgm1, dgm2)
